# baseline (speedup 1.0000x reference)
import functools

import jax
import jax.numpy as jnp
from jax import lax
from jax.experimental import pallas as pl
from jax.experimental.pallas import tpu as pltpu
from jax.experimental.pallas import tpu_sc as plsc

_L = 100
_D = 128
_LPAD = 128


def _sc_gather(idx_pad, table):
    mesh = plsc.VectorSubcoreMesh(core_axis_name="c", subcore_axis_name="s")

    @functools.partial(
        pl.kernel,
        mesh=mesh,
        out_type=jax.ShapeDtypeStruct((_LPAD, _D), jnp.float32),
        scratch_types=[
            pltpu.VMEM((_LPAD,), jnp.int32),
            pltpu.VMEM((_LPAD, _D), jnp.float32),
            pltpu.SemaphoreType.DMA,
        ],
    )
    def gather_kernel(idx_hbm, table_hbm, out_hbm, idx_v, rows_v, sem):
        wid = lax.axis_index("s") * 2 + lax.axis_index("c")

        @pl.when(wid == 0)
        def _():
            pltpu.sync_copy(idx_hbm, idx_v)
            pltpu.async_copy(table_hbm.at[idx_v], rows_v, sem).wait()
            pltpu.sync_copy(rows_v, out_hbm)

    return gather_kernel(idx_pad, table)


_B = 1024
_SS = 25


def _add_body(e_ref, x_ref, o_ref):
    o_ref[...] = x_ref[...] + e_ref[...]


def _tc_add_t(x_t, embed3):
    nb = _L // _SS
    return pl.pallas_call(
        _add_body,
        grid=(nb,),
        in_specs=[
            pl.BlockSpec((_SS, 1, _D), lambda i: (i, 0, 0)),
            pl.BlockSpec((_SS, _B, _D), lambda i: (i, 0, 0)),
        ],
        out_specs=pl.BlockSpec((_SS, _B, _D), lambda i: (i, 0, 0)),
        out_shape=jax.ShapeDtypeStruct(x_t.shape, x_t.dtype),
    )(embed3, x_t)


def kernel(x, variable_idx, variable_embedding):
    idx = variable_idx.astype(jnp.int32)
    idx_pad = jnp.pad(idx, (0, _LPAD - _L))
    embed_pad = _sc_gather(idx_pad, variable_embedding)
    embed3 = embed_pad[:_L].reshape(_L, 1, _D)
    x_t = jnp.transpose(x, (1, 0, 2))
    out_t = _tc_add_t(x_t, embed3)
    return jnp.transpose(out_t, (1, 0, 2))
